# baseline (speedup 1.0000x reference)
import functools
import math

import jax
import jax.numpy as jnp
from jax import lax
from jax.experimental import pallas as pl
from jax.experimental.pallas import tpu as pltpu
from jax.experimental.pallas import tpu_sc as plsc

VOCAB = 1000000
D = 64
SCALE = math.sqrt(D)

NC = 2
NS = 16
NW = NC * NS

SEQ = 200
CH = 128
NB = 2


def _body(idx_hbm, table_hbm, out_hbm, idx_v, rows_g, rows_s, g0, g1, s0, s1):
    wid = lax.axis_index("s") * NC + lax.axis_index("c")
    gsem = (g0, g1)
    ssem = (s0, s1)

    pltpu.sync_copy(idx_hbm.at[:, pl.ds(wid * CH, CH)], idx_v)

    lanes = lax.iota(jnp.int32, 16)
    rowoff = [(lanes + k * 16) * CH for k in range(D // 16)]

    def issue_gather(j, b):
        pltpu.async_copy(table_hbm.at[idx_v.at[j]], rows_g.at[b], gsem[b])

    def wait_gather(b):
        pltpu.make_async_copy(table_hbm.at[pl.ds(0, CH)], rows_g.at[b], gsem[b]).wait()

    def issue_scatter(j, b):
        for d8 in range(D // 8):
            pltpu.async_copy(
                rows_s.at[b, pl.ds(d8 * 1024, 1024)], out_hbm.at[j, d8, wid], ssem[b]
            )

    def wait_scatter(b):
        for d8 in range(D // 8):
            pltpu.make_async_copy(
                rows_s.at[b, pl.ds(d8 * 1024, 1024)], out_hbm.at[0, d8, 0], ssem[b]
            ).wait()

    def transform(b):
        @plsc.parallel_loop(0, CH, 1, unroll=4)
        def _(t):
            for k in range(D // 16):
                vals = rows_g[b, t, pl.ds(k * 16, 16)] * SCALE
                plsc.store_scatter(rows_s.at[b], [rowoff[k] + t], vals)

    for b in range(NB):
        issue_gather(b, b)
    for b in range(NB):
        wait_gather(b)
        transform(b)
        issue_scatter(b, b)
        issue_gather(b + NB, b)

    def outer(g, carry):
        for b in range(NB):
            j = NB + g * NB + b
            wait_gather(b)
            wait_scatter(b)
            transform(b)
            issue_scatter(j, b)
            issue_gather(j + NB, b)
        return carry

    lax.fori_loop(0, (SEQ - 2 * NB) // NB, outer, 0)

    for b in range(NB):
        j = SEQ - NB + b
        wait_gather(b)
        wait_scatter(b)
        transform(b)
        issue_scatter(j, b)
    for b in range(NB):
        wait_scatter(b)


@jax.jit
def _lookup(x_idx, weight):
    mesh = plsc.VectorSubcoreMesh(core_axis_name="c", subcore_axis_name="s")
    f = pl.kernel(
        _body,
        mesh=mesh,
        out_type=jax.ShapeDtypeStruct((SEQ, D // 8, NW, 1024), jnp.float32),
        scratch_types=[
            pltpu.VMEM((SEQ, CH), jnp.int32),
            pltpu.VMEM((NB, CH, D), jnp.float32),
            pltpu.VMEM((NB, D * CH), jnp.float32),
            pltpu.SemaphoreType.DMA,
            pltpu.SemaphoreType.DMA,
            pltpu.SemaphoreType.DMA,
            pltpu.SemaphoreType.DMA,
        ],
        compiler_params=pltpu.CompilerParams(
            use_tc_tiling_on_sc=False, needs_layout_passes=False
        ),
    )
    return f(x_idx, weight)


def kernel(x, weight):
    xT = x.astype(jnp.int32).T
    out6 = _lookup(xT, weight)
    return (
        out6.reshape(SEQ, D // 8, NW, 8, CH)
        .transpose(2, 4, 0, 1, 3)
        .reshape(4096, SEQ, D)
    )


# speedup vs baseline: 1.0047x; 1.0047x over previous
import functools
import math

import jax
import jax.numpy as jnp
from jax import lax
from jax.experimental import pallas as pl
from jax.experimental.pallas import tpu as pltpu
from jax.experimental.pallas import tpu_sc as plsc

VOCAB = 1000000
D = 64
DP = 128
SCALE = math.sqrt(D)

NC = 2
NS = 16
NW = NC * NS

SEQ = 200
CH = 128
NB = 2


def _body(idx_hbm, table_hbm, out_hbm, idx_v, rows_g, rows_s, g0, g1, s0, s1):
    wid = lax.axis_index("s") * NC + lax.axis_index("c")
    gsem = (g0, g1)
    ssem = (s0, s1)

    pltpu.sync_copy(idx_hbm.at[:, pl.ds(wid * CH, CH)], idx_v)

    def issue_gather(j, b):
        pltpu.async_copy(table_hbm.at[idx_v.at[j]], rows_g.at[b], gsem[b])

    def wait_gather(b):
        pltpu.make_async_copy(table_hbm.at[pl.ds(0, CH)], rows_g.at[b], gsem[b]).wait()

    def issue_scatter(j, b):
        pltpu.async_copy(rows_s.at[b], out_hbm.at[wid, j], ssem[b])

    def wait_scatter(b):
        pltpu.make_async_copy(rows_s.at[b], out_hbm.at[wid, 0], ssem[b]).wait()

    def scale(b):
        @plsc.parallel_loop(0, CH, 1, unroll=4)
        def _(t):
            for k in range(DP // 16):
                sl = pl.ds(k * 16, 16)
                rows_s[b, t, sl] = rows_g[b, t, sl] * SCALE

    for b in range(NB):
        issue_gather(b, b)
    for b in range(NB):
        wait_gather(b)
        scale(b)
        issue_scatter(b, b)
        issue_gather(b + NB, b)

    def outer(g, carry):
        for b in range(NB):
            j = NB + g * NB + b
            wait_gather(b)
            wait_scatter(b)
            scale(b)
            issue_scatter(j, b)
            issue_gather(j + NB, b)
        return carry

    lax.fori_loop(0, (SEQ - 2 * NB) // NB, outer, 0)

    for b in range(NB):
        j = SEQ - NB + b
        wait_gather(b)
        wait_scatter(b)
        scale(b)
        issue_scatter(j, b)
    for b in range(NB):
        wait_scatter(b)


@jax.jit
def _lookup(x_idx, table):
    mesh = plsc.VectorSubcoreMesh(core_axis_name="c", subcore_axis_name="s")
    f = pl.kernel(
        _body,
        mesh=mesh,
        out_type=jax.ShapeDtypeStruct((NW, SEQ, CH, DP), jnp.float32),
        scratch_types=[
            pltpu.VMEM((SEQ, CH), jnp.int32),
            pltpu.VMEM((NB, CH, DP), jnp.float32),
            pltpu.VMEM((NB, CH, DP), jnp.float32),
            pltpu.SemaphoreType.DMA,
            pltpu.SemaphoreType.DMA,
            pltpu.SemaphoreType.DMA,
            pltpu.SemaphoreType.DMA,
        ],
        compiler_params=pltpu.CompilerParams(
            use_tc_tiling_on_sc=False, needs_layout_passes=False
        ),
    )
    return f(x_idx, table)


def kernel(x, weight):
    xT = x.astype(jnp.int32).T
    wp = jnp.pad(weight, ((0, 0), (0, DP - D)))
    out7 = _lookup(xT, wp)
    emb = out7.transpose(0, 2, 1, 3).reshape(4096, SEQ, DP)
    return emb[:, :, :D]
